# baseline (speedup 1.0000x reference)
import functools

import jax
import jax.numpy as jnp
from jax import lax
from jax.experimental import pallas as pl
from jax.experimental.pallas import tpu as pltpu
from jax.experimental.pallas import tpu_sc as plsc

V = 1000000
D = 32
B = 4096
S = 200
N = B * S

NC = 2
NS = 16
NW = NC * NS

BW = B // NW
SCH = 5
CHUNKS = S // SCH
ROWS = SCH * BW

_mesh = plsc.VectorSubcoreMesh(core_axis_name="c", subcore_axis_name="s")


@functools.partial(
    pl.kernel,
    out_type=(
        jax.ShapeDtypeStruct((S, D, B), jnp.float32),
        jax.ShapeDtypeStruct((S, D, B), jnp.float32),
    ),
    mesh=_mesh,
    compiler_params=pltpu.CompilerParams(use_tc_tiling_on_sc=False, needs_layout_passes=False),
    scratch_types=[
        pltpu.VMEM((SCH, BW), jnp.int32),
        pltpu.VMEM((2 * ROWS, D), jnp.float32),
        pltpu.VMEM((SCH, D, BW + 1), jnp.float32),
        pltpu.VMEM((SCH, D, BW + 1), jnp.float32),
        pltpu.SemaphoreType.DMA,
        pltpu.SemaphoreType.DMA,
        pltpu.SemaphoreType.DMA,
        pltpu.SemaphoreType.DMA,
    ],
)
def _sc_gather(xt, wr, wi, out_r, out_i, idx_v, rows_v, obuf_r, obuf_i,
               rsem, isem, orsem, oisem):
    wid = lax.axis_index("s") * NC + lax.axis_index("c")
    b0 = wid * BW

    lane = lax.iota(jnp.int32, 16)

    def transpose_block(obuf, base):
        def col(bb, carry):
            bv = lane * 0 + bb
            for j in range(SCH):
                jv = jnp.full((16,), j, jnp.int32)
                for dh in range(D // 16):
                    vals = rows_v[base + j * BW + bb, pl.ds(dh * 16, 16)]
                    plsc.store_scatter(obuf, [jv, dh * 16 + lane, bv], vals)
            return carry
        lax.fori_loop(0, BW, col, 0)

    def gather_rows(w, base, sem):
        for j in range(SCH):
            pltpu.async_copy(w.at[idx_v.at[j]],
                             rows_v.at[pl.ds(base + j * BW, BW)], sem)

    def drain_rows(w, base, sem):
        for j in range(SCH):
            pltpu.make_async_copy(w.at[idx_v.at[j]],
                                  rows_v.at[pl.ds(base + j * BW, BW)],
                                  sem).wait()

    def out_copy(obuf, out, s0, sem):
        return pltpu.make_async_copy(
            obuf.at[:, :, pl.ds(0, BW)],
            out.at[pl.ds(s0, SCH), :, pl.ds(b0, BW)], sem)

    def chunk(c, carry):
        s0 = c * SCH
        pltpu.sync_copy(xt.at[pl.ds(s0, SCH), pl.ds(b0, BW)], idx_v)
        gather_rows(wr, 0, rsem)
        gather_rows(wi, ROWS, isem)

        @pl.when(c > 0)
        def _():
            out_copy(obuf_r, out_r, s0 - SCH, orsem).wait()
            out_copy(obuf_i, out_i, s0 - SCH, oisem).wait()

        drain_rows(wr, 0, rsem)
        transpose_block(obuf_r, 0)
        out_copy(obuf_r, out_r, s0, orsem).start()
        drain_rows(wi, ROWS, isem)
        transpose_block(obuf_i, ROWS)
        out_copy(obuf_i, out_i, s0, oisem).start()
        return carry

    lax.fori_loop(0, CHUNKS, chunk, 0)
    s_last = (CHUNKS - 1) * SCH
    out_copy(obuf_r, out_r, s_last, orsem).wait()
    out_copy(obuf_i, out_i, s_last, oisem).wait()


def kernel(x, W_real, W_imag):
    xt = x.T
    r, i = _sc_gather(xt, W_real, W_imag)
    r3 = jnp.transpose(r, (2, 0, 1))
    i3 = jnp.transpose(i, (2, 0, 1))
    return lax.complex(r3, i3)


# speedup vs baseline: 1.0258x; 1.0258x over previous
import functools

import jax
import jax.numpy as jnp
from jax import lax
from jax.experimental import pallas as pl
from jax.experimental.pallas import tpu as pltpu
from jax.experimental.pallas import tpu_sc as plsc

V = 1000000
D = 32
B = 4096
S = 200
N = B * S

NC = 2
NS = 16
NW = NC * NS

BW = B // NW
SCH = 5
CHUNKS = S // SCH
ROWS = SCH * BW

_mesh = plsc.VectorSubcoreMesh(core_axis_name="c", subcore_axis_name="s")


@functools.partial(
    pl.kernel,
    out_type=(
        jax.ShapeDtypeStruct((S, D, B), jnp.float32),
        jax.ShapeDtypeStruct((S, D, B), jnp.float32),
    ),
    mesh=_mesh,
    compiler_params=pltpu.CompilerParams(use_tc_tiling_on_sc=False, needs_layout_passes=False),
    scratch_types=[
        pltpu.VMEM((SCH, BW), jnp.int32),
        pltpu.VMEM((SCH, BW), jnp.int32),
        pltpu.VMEM((2 * ROWS, D), jnp.float32),
        pltpu.VMEM((2 * ROWS, D), jnp.float32),
        pltpu.VMEM((SCH, D, BW + 1), jnp.float32),
        pltpu.VMEM((SCH, D, BW + 1), jnp.float32),
        pltpu.SemaphoreType.DMA,
        pltpu.SemaphoreType.DMA,
        pltpu.SemaphoreType.DMA,
        pltpu.SemaphoreType.DMA,
    ],
)
def _sc_gather(xt, wr, wi, out_r, out_i, idx_a, idx_b, rows_a, rows_b,
               obuf_r, obuf_i, rsem, isem, orsem, oisem):
    wid = lax.axis_index("s") * NC + lax.axis_index("c")
    b0 = wid * BW

    lane = lax.iota(jnp.int32, 16)

    def transpose_block(obuf, rows, base):
        def col(bb, carry):
            bv = lane * 0 + bb
            for j in range(SCH):
                jv = jnp.full((16,), j, jnp.int32)
                for dh in range(D // 16):
                    vals = rows[base + j * BW + bb, pl.ds(dh * 16, 16)]
                    plsc.store_scatter(obuf, [jv, dh * 16 + lane, bv], vals)
            return carry
        lax.fori_loop(0, BW, col, 0)

    def stage_fire(c, idx, rows):
        s0 = c * SCH
        pltpu.sync_copy(xt.at[pl.ds(s0, SCH), pl.ds(b0, BW)], idx)
        for j in range(SCH):
            pltpu.async_copy(wr.at[idx.at[j]],
                             rows.at[pl.ds(j * BW, BW)], rsem)
            pltpu.async_copy(wi.at[idx.at[j]],
                             rows.at[pl.ds(ROWS + j * BW, BW)], isem)

    def drain_rows(w, idx, rows, base, sem):
        for j in range(SCH):
            pltpu.make_async_copy(w.at[idx.at[j]],
                                  rows.at[pl.ds(base + j * BW, BW)],
                                  sem).wait()

    def out_copy(obuf, out, s0, sem):
        return pltpu.make_async_copy(
            obuf.at[:, :, pl.ds(0, BW)],
            out.at[pl.ds(s0, SCH), :, pl.ds(b0, BW)], sem)

    def consume(c, idx, rows):
        s0 = c * SCH
        @pl.when(c > 0)
        def _():
            out_copy(obuf_r, out_r, s0 - SCH, orsem).wait()
            out_copy(obuf_i, out_i, s0 - SCH, oisem).wait()
        drain_rows(wr, idx, rows, 0, rsem)
        transpose_block(obuf_r, rows, 0)
        out_copy(obuf_r, out_r, s0, orsem).start()
        drain_rows(wi, idx, rows, ROWS, isem)
        transpose_block(obuf_i, rows, ROWS)
        out_copy(obuf_i, out_i, s0, oisem).start()

    stage_fire(0, idx_a, rows_a)

    def two(it, carry):
        c0 = it * 2
        stage_fire(c0 + 1, idx_b, rows_b)
        consume(c0, idx_a, rows_a)

        @pl.when(c0 + 2 < CHUNKS)
        def _():
            stage_fire(c0 + 2, idx_a, rows_a)
        consume(c0 + 1, idx_b, rows_b)
        return carry

    lax.fori_loop(0, CHUNKS // 2, two, 0)
    s_last = (CHUNKS - 1) * SCH
    out_copy(obuf_r, out_r, s_last, orsem).wait()
    out_copy(obuf_i, out_i, s_last, oisem).wait()


def kernel(x, W_real, W_imag):
    xt = x.T
    r, i = _sc_gather(xt, W_real, W_imag)
    r3 = jnp.transpose(r, (2, 0, 1))
    i3 = jnp.transpose(i, (2, 0, 1))
    return lax.complex(r3, i3)
